# baseline (speedup 1.0000x reference)
import functools

import jax
import jax.numpy as jnp
from jax import lax
from jax.experimental import pallas as pl
from jax.experimental.pallas import tpu as pltpu
from jax.experimental.pallas import tpu_sc as plsc

_B = 1024
_S = 200
_DIM = 128
_HALF = _DIM // 2
_BASE = 10000

_NC = 2
_NS = 16
_NW = _NC * _NS
_BPW = _B // _NW
_GCHUNK = _S // 2


def _sincos():
    inv_freq = 1.0 / (_BASE ** (jnp.arange(0, _HALF, dtype=jnp.float32) / _HALF))
    angles = jnp.arange(_S, dtype=jnp.float32)[:, None] * inv_freq[None, :]
    return jnp.sin(angles), jnp.cos(angles)


def _body(ids_ref, table_ref, sin_ref, cos_ref, out_ref,
          idx_v, rows_v, sin_v, cos_v, gsem0, gsem1, wsem0, wsem1):
    wid = lax.axis_index("s") * _NC + lax.axis_index("c")
    base = wid * _BPW

    pltpu.sync_copy(sin_ref, sin_v)
    pltpu.sync_copy(cos_ref, cos_v)

    gsem = (gsem0, gsem1)
    wsem = (wsem0, wsem1)

    def start_gather(k, p):
        pltpu.sync_copy(ids_ref.at[base + k], idx_v.at[p])
        return [
            pltpu.async_copy(table_ref.at[idx_v.at[p, c]],
                             rows_v.at[p, pl.ds(c * _GCHUNK, _GCHUNK)],
                             gsem[p])
            for c in range(2)
        ]

    def compute(p):
        def row_body(i, c2):
            for j in range(_HALF // 16):
                lo = pl.ds(j * 16, 16)
                hi = pl.ds(_HALF + j * 16, 16)
                t1 = rows_v[p, i, lo]
                t2 = rows_v[p, i, hi]
                cosv = cos_v[i, lo]
                sinv = sin_v[i, lo]
                rows_v[p, i, lo] = t1 * cosv - t2 * sinv
                rows_v[p, i, hi] = t1 * sinv + t2 * cosv
            return c2

        lax.fori_loop(0, _S, row_body, 0, unroll=2)

    gcur = start_gather(0, 0)
    wpend = [None, None]
    for k in range(_BPW):
        p = k & 1
        gnext = None
        if k + 1 < _BPW:
            if wpend[1 - p] is not None:
                wpend[1 - p].wait()
                wpend[1 - p] = None
            gnext = start_gather(k + 1, 1 - p)
        for cp in gcur:
            cp.wait()
        gcur = gnext
        compute(p)
        wpend[p] = pltpu.async_copy(rows_v.at[p], out_ref.at[base + k],
                                    wsem[p])
    for w in wpend:
        if w is not None:
            w.wait()


@jax.jit
def _run(ids2, table, sin, cos):
    mesh = plsc.VectorSubcoreMesh(core_axis_name="c", subcore_axis_name="s",
                                  num_cores=_NC, num_subcores=_NS)
    f = pl.kernel(
        _body,
        out_type=jax.ShapeDtypeStruct((_B, _S, _DIM), jnp.float32),
        mesh=mesh,
        scratch_types=[
            pltpu.VMEM((2, 2, _GCHUNK), jnp.int32),
            pltpu.VMEM((2, _S, _DIM), jnp.float32),
            pltpu.VMEM((_S, _HALF), jnp.float32),
            pltpu.VMEM((_S, _HALF), jnp.float32),
            pltpu.SemaphoreType.DMA,
            pltpu.SemaphoreType.DMA,
            pltpu.SemaphoreType.DMA,
            pltpu.SemaphoreType.DMA,
        ],
    )
    return f(ids2, table, sin, cos)


def kernel(ids, table):
    sin, cos = _sincos()
    ids2 = ids.reshape(_B, 2, _GCHUNK)
    return _run(ids2, table, sin, cos)


# speedup vs baseline: 1.3862x; 1.3862x over previous
import functools

import jax
import jax.numpy as jnp
from jax import lax
from jax.experimental import pallas as pl
from jax.experimental.pallas import tpu as pltpu
from jax.experimental.pallas import tpu_sc as plsc

_B = 1024
_S = 200
_DIM = 128
_HALF = _DIM // 2
_BASE = 10000

_NC = 2
_NS = 16
_NW = _NC * _NS
_BPW = _B // _NW
_GCHUNK = _S // 2


def _sincos():
    inv_freq = 1.0 / (_BASE ** (jnp.arange(0, _HALF, dtype=jnp.float32) / _HALF))
    angles = jnp.arange(_S, dtype=jnp.float32)[:, None] * inv_freq[None, :]
    return jnp.sin(angles), jnp.cos(angles)


def _body(ids_ref, table_ref, sin_ref, cos_ref, out_ref,
          idx_v, rows_v, sin_v, cos_v, gsem0, gsem1, wsem0, wsem1):
    wid = lax.axis_index("s") * _NC + lax.axis_index("c")
    base = wid * _BPW

    pltpu.sync_copy(sin_ref, sin_v)
    pltpu.sync_copy(cos_ref, cos_v)

    gsem = (gsem0, gsem1)
    wsem = (wsem0, wsem1)

    def start_gather(k, p):
        pltpu.sync_copy(ids_ref.at[base + k], idx_v.at[p])
        return [
            pltpu.async_copy(table_ref.at[idx_v.at[p, c]],
                             rows_v.at[p, pl.ds(c * _GCHUNK, _GCHUNK)],
                             gsem[p])
            for c in range(2)
        ]

    def compute(p):
        @plsc.parallel_loop(0, _S, step=1, unroll=2)
        def row_body(i):
            for j in range(_HALF // 16):
                lo = pl.ds(j * 16, 16)
                hi = pl.ds(_HALF + j * 16, 16)
                t1 = rows_v[p, i, lo]
                t2 = rows_v[p, i, hi]
                cosv = cos_v[i, lo]
                sinv = sin_v[i, lo]
                rows_v[p, i, lo] = t1 * cosv - t2 * sinv
                rows_v[p, i, hi] = t1 * sinv + t2 * cosv

    gcur = start_gather(0, 0)
    wpend = [None, None]
    for k in range(_BPW):
        p = k & 1
        gnext = None
        if k + 1 < _BPW:
            if wpend[1 - p] is not None:
                wpend[1 - p].wait()
                wpend[1 - p] = None
            gnext = start_gather(k + 1, 1 - p)
        for cp in gcur:
            cp.wait()
        gcur = gnext
        compute(p)
        wpend[p] = pltpu.async_copy(rows_v.at[p], out_ref.at[base + k],
                                    wsem[p])
    for w in wpend:
        if w is not None:
            w.wait()


@jax.jit
def _run(ids2, table, sin, cos):
    mesh = plsc.VectorSubcoreMesh(core_axis_name="c", subcore_axis_name="s",
                                  num_cores=_NC, num_subcores=_NS)
    f = pl.kernel(
        _body,
        out_type=jax.ShapeDtypeStruct((_B, _S, _DIM), jnp.float32),
        mesh=mesh,
        scratch_types=[
            pltpu.VMEM((2, 2, _GCHUNK), jnp.int32),
            pltpu.VMEM((2, _S, _DIM), jnp.float32),
            pltpu.VMEM((_S, _HALF), jnp.float32),
            pltpu.VMEM((_S, _HALF), jnp.float32),
            pltpu.SemaphoreType.DMA,
            pltpu.SemaphoreType.DMA,
            pltpu.SemaphoreType.DMA,
            pltpu.SemaphoreType.DMA,
        ],
    )
    return f(ids2, table, sin, cos)


def kernel(ids, table):
    sin, cos = _sincos()
    ids2 = ids.reshape(_B, 2, _GCHUNK)
    return _run(ids2, table, sin, cos)
